# baseline (speedup 1.0000x reference)
import functools

import jax
import jax.numpy as jnp
from jax import lax
from jax.experimental import pallas as pl
from jax.experimental.pallas import tpu as pltpu
from jax.experimental.pallas import tpu_sc as plsc

_BATCH = 4096
_VOCAB = 1000
_DIM = 128

_NC = 2
_NS = 16
_SCB = 2048
_TCB = _BATCH - _SCB
_BB = 512
_BPC = _SCB // _NC
_NCG = _BPC // 16
_OPT = _BPC // _NS
_TROWS = _VOCAB // 8
_NBUF = 2


def _row_sums_body(t_ref, o_ref):
    o_ref[...] = jnp.sum(t_ref[...], axis=1)


def _row_sums(table):
    return pl.pallas_call(
        _row_sums_body,
        out_shape=jax.ShapeDtypeStruct((_VOCAB,), jnp.float32),
    )(table)


def _sc_body(in_hbm, rs_hbm, out_hbm, rsbuf, inbufA, inbufB, acc_s, acc_c,
             outbuf, tmp_s, tmp_c, sh_s, sh_c, sem0, sem1):
    cid = lax.axis_index("c")
    sid = lax.axis_index("s")
    col0 = cid * _BPC
    ntr = jnp.where(sid < _TROWS - 7 * _NS, 8, 7)

    zf = jnp.zeros((16,), jnp.float32)
    zi = jnp.zeros((16,), jnp.int32)
    onef = jnp.ones((16,), jnp.float32)
    lane = lax.iota(jnp.int32, 16)

    pltpu.sync_copy(rs_hbm, rsbuf)

    def zero_body(i, carry):
        acc_s[pl.ds(i * 16, 16)] = zf
        acc_c[pl.ds(i * 16, 16)] = zi
        return carry

    lax.fori_loop(0, _NCG, zero_body, 0)

    inbufs = (inbufA, inbufB)
    sems = (sem0, sem1)

    def _fire(t, b):
        tr = sid + t * _NS
        pltpu.make_async_copy(
            in_hbm.at[pl.ds(tr * 8, 8), pl.ds(col0, _BPC)],
            inbufs[b],
            sems[b],
        ).start()

    def _drain(b):
        pltpu.make_async_copy(
            in_hbm.at[pl.ds(0, 8), pl.ds(col0, _BPC)],
            inbufs[b],
            sems[b],
        ).wait()

    def _compute(t, b):
        ib = inbufs[b]
        tr8 = (sid + t * _NS) * 8
        rsvs = [
            plsc.load_gather(rsbuf, [jnp.full((16,), tr8 + g, jnp.int32)])
            for g in range(8)
        ]

        def c_body(c):
            a_s = acc_s[pl.ds(c * 16, 16)]
            a_c = acc_c[pl.ds(c * 16, 16)]
            for g in range(8):
                x = ib[g, pl.ds(c * 16, 16)]
                a_c = a_c + x
                a_s = a_s + x.astype(jnp.float32) * rsvs[g]
            acc_s[pl.ds(c * 16, 16)] = a_s
            acc_c[pl.ds(c * 16, 16)] = a_c

        plsc.parallel_loop(0, _NCG, 1, unroll=2)(c_body)

    for b in range(_NBUF):
        _fire(b, b)

    def pair_body(p, carry):
        for b in range(_NBUF):
            t = p * _NBUF + b

            @pl.when(t < ntr)
            def _():
                _drain(b)
                _compute(t, b)

                @pl.when(t + _NBUF < ntr)
                def _():
                    _fire(t + _NBUF, b)

        return carry

    lax.fori_loop(0, 4, pair_body, 0)

    pltpu.sync_copy(acc_s, sh_s.at[sid])
    pltpu.sync_copy(acc_c, sh_c.at[sid])
    plsc.subcore_barrier()

    pltpu.sync_copy(sh_s, tmp_s)
    pltpu.sync_copy(sh_c, tmp_c)

    inv_d = jnp.float32(1.0 / _DIM)
    vecs = []
    for cg in range(_OPT // 16):
        tot_s = zf
        tot_c = zi
        for t in range(_NS):
            tot_s = tot_s + tmp_s[t, pl.ds(sid * _OPT + cg * 16, 16)]
            tot_c = tot_c + tmp_c[t, pl.ds(sid * _OPT + cg * 16, 16)]
        vecs.append(
            tot_s * inv_d
            / jnp.maximum(tot_c.astype(jnp.float32), onef))
    rows = [cg * 16 + lane for cg in range(_OPT // 16)]

    def d_body(d, carry):
        dsplat = jnp.full((16,), d, jnp.int32)
        for cg in range(_OPT // 16):
            plsc.store_scatter(outbuf, [rows[cg], dsplat], vecs[cg])
        return carry

    lax.fori_loop(0, _DIM, d_body, 0)

    base = col0 + sid * _OPT
    pltpu.sync_copy(outbuf, out_hbm.at[pl.ds(base, _OPT)])


def _sc_main(inp_t, rs):
    mesh = plsc.VectorSubcoreMesh(core_axis_name="c", subcore_axis_name="s")
    kern = functools.partial(
        pl.kernel,
        out_type=jax.ShapeDtypeStruct((_SCB, _DIM), jnp.float32),
        mesh=mesh,
        compiler_params=pltpu.CompilerParams(needs_layout_passes=False),
        scratch_types=[
            pltpu.VMEM((_VOCAB,), jnp.float32),
            pltpu.VMEM((8, _BPC), jnp.int32),
            pltpu.VMEM((8, _BPC), jnp.int32),
            pltpu.VMEM((_BPC,), jnp.float32),
            pltpu.VMEM((_BPC,), jnp.int32),
            pltpu.VMEM((_OPT, _DIM), jnp.float32),
            pltpu.VMEM((_NS, _BPC), jnp.float32),
            pltpu.VMEM((_NS, _BPC), jnp.int32),
            pltpu.VMEM_SHARED((_NS, _BPC), jnp.float32),
            pltpu.VMEM_SHARED((_NS, _BPC), jnp.int32),
            pltpu.SemaphoreType.DMA,
            pltpu.SemaphoreType.DMA,
        ],
    )(_sc_body)
    return kern(inp_t, rs)


def _tc_body(xt_ref, tab_ref, o_ref):
    x = xt_ref[...]
    rs = jnp.sum(tab_ref[...], axis=1)
    m = x != 0
    sel = jnp.sum(jnp.where(m, rs[:, None], jnp.float32(0.0)), axis=0)
    cnt = jnp.sum(m, axis=0).astype(jnp.float32)
    scal = sel / (jnp.maximum(cnt, 1.0) * jnp.float32(_DIM))
    o_ref[...] = jnp.broadcast_to(scal[:, None], (_BB, _DIM))


def _tc_part(xt, table):
    return pl.pallas_call(
        _tc_body,
        grid=(_TCB // _BB,),
        in_specs=[
            pl.BlockSpec((_VOCAB, _BB), lambda i: (0, _SCB // _BB + i)),
            pl.BlockSpec((_VOCAB, _DIM), lambda i: (0, 0)),
        ],
        out_specs=pl.BlockSpec((_BB, _DIM), lambda i: (i, 0)),
        out_shape=jax.ShapeDtypeStruct((_TCB, _DIM), jnp.float32),
    )(xt, table)


def kernel(input, table):
    xt = input.T
    rs = _row_sums(table)
    sc_out = _sc_main(xt, rs)
    tc_out = _tc_part(xt, table)
    return jnp.concatenate([sc_out, tc_out], axis=0)


# speedup vs baseline: 1.1421x; 1.1421x over previous
import functools

import jax
import jax.numpy as jnp
from jax import lax
from jax.experimental import pallas as pl
from jax.experimental.pallas import tpu as pltpu
from jax.experimental.pallas import tpu_sc as plsc

_BATCH = 4096
_VOCAB = 1000
_DIM = 128

_NC = 2
_NS = 16
_SCB = 1024
_TCB = _BATCH - _SCB
_BB = 512
_BPC = _SCB // _NC
_NCG = _BPC // 16
_OPT = _BPC // _NS
_TROWS = _VOCAB // 8
_NBUF = 2


def _row_sums_body(t_ref, o_ref):
    o_ref[...] = jnp.sum(t_ref[...], axis=1)


def _row_sums(table):
    return pl.pallas_call(
        _row_sums_body,
        out_shape=jax.ShapeDtypeStruct((_VOCAB,), jnp.float32),
    )(table)


def _sc_body(in_hbm, rs_hbm, out_hbm, rsbuf, inbufA, inbufB, acc_s, acc_c,
             outbuf, tmp_s, tmp_c, sh_s, sh_c, sem0, sem1):
    cid = lax.axis_index("c")
    sid = lax.axis_index("s")
    col0 = cid * _BPC
    ntr = jnp.where(sid < _TROWS - 7 * _NS, 8, 7)

    zf = jnp.zeros((16,), jnp.float32)
    zi = jnp.zeros((16,), jnp.int32)
    onef = jnp.ones((16,), jnp.float32)
    lane = lax.iota(jnp.int32, 16)

    pltpu.sync_copy(rs_hbm, rsbuf)

    def zero_body(i, carry):
        acc_s[pl.ds(i * 16, 16)] = zf
        acc_c[pl.ds(i * 16, 16)] = zi
        return carry

    lax.fori_loop(0, _NCG, zero_body, 0)

    inbufs = (inbufA, inbufB)
    sems = (sem0, sem1)

    def _fire(t, b):
        tr = sid + t * _NS
        pltpu.make_async_copy(
            in_hbm.at[pl.ds(tr * 8, 8), pl.ds(col0, _BPC)],
            inbufs[b],
            sems[b],
        ).start()

    def _drain(b):
        pltpu.make_async_copy(
            in_hbm.at[pl.ds(0, 8), pl.ds(col0, _BPC)],
            inbufs[b],
            sems[b],
        ).wait()

    def _compute(t, b):
        ib = inbufs[b]
        tr8 = (sid + t * _NS) * 8
        rsvs = [
            plsc.load_gather(rsbuf, [jnp.full((16,), tr8 + g, jnp.int32)])
            for g in range(8)
        ]

        def c_body(c):
            a_s = acc_s[pl.ds(c * 16, 16)]
            a_c = acc_c[pl.ds(c * 16, 16)]
            for g in range(8):
                x = ib[g, pl.ds(c * 16, 16)]
                a_c = a_c + x
                a_s = a_s + x.astype(jnp.float32) * rsvs[g]
            acc_s[pl.ds(c * 16, 16)] = a_s
            acc_c[pl.ds(c * 16, 16)] = a_c

        plsc.parallel_loop(0, _NCG, 1, unroll=2)(c_body)

    for b in range(_NBUF):
        _fire(b, b)

    def pair_body(p, carry):
        for b in range(_NBUF):
            t = p * _NBUF + b

            @pl.when(t < ntr)
            def _():
                _drain(b)
                _compute(t, b)

                @pl.when(t + _NBUF < ntr)
                def _():
                    _fire(t + _NBUF, b)

        return carry

    lax.fori_loop(0, 4, pair_body, 0)

    pltpu.sync_copy(acc_s, sh_s.at[sid])
    pltpu.sync_copy(acc_c, sh_c.at[sid])
    plsc.subcore_barrier()

    pltpu.sync_copy(sh_s, tmp_s)
    pltpu.sync_copy(sh_c, tmp_c)

    inv_d = jnp.float32(1.0 / _DIM)
    vecs = []
    for cg in range(_OPT // 16):
        tot_s = zf
        tot_c = zi
        for t in range(_NS):
            tot_s = tot_s + tmp_s[t, pl.ds(sid * _OPT + cg * 16, 16)]
            tot_c = tot_c + tmp_c[t, pl.ds(sid * _OPT + cg * 16, 16)]
        vecs.append(
            tot_s * inv_d
            / jnp.maximum(tot_c.astype(jnp.float32), onef))
    rows = [cg * 16 + lane for cg in range(_OPT // 16)]

    def d_body(d, carry):
        dsplat = jnp.full((16,), d, jnp.int32)
        for cg in range(_OPT // 16):
            plsc.store_scatter(outbuf, [rows[cg], dsplat], vecs[cg])
        return carry

    lax.fori_loop(0, _DIM, d_body, 0)

    base = col0 + sid * _OPT
    pltpu.sync_copy(outbuf, out_hbm.at[pl.ds(base, _OPT)])


def _sc_main(inp_t, rs):
    mesh = plsc.VectorSubcoreMesh(core_axis_name="c", subcore_axis_name="s")
    kern = functools.partial(
        pl.kernel,
        out_type=jax.ShapeDtypeStruct((_SCB, _DIM), jnp.float32),
        mesh=mesh,
        compiler_params=pltpu.CompilerParams(needs_layout_passes=False),
        scratch_types=[
            pltpu.VMEM((_VOCAB,), jnp.float32),
            pltpu.VMEM((8, _BPC), jnp.int32),
            pltpu.VMEM((8, _BPC), jnp.int32),
            pltpu.VMEM((_BPC,), jnp.float32),
            pltpu.VMEM((_BPC,), jnp.int32),
            pltpu.VMEM((_OPT, _DIM), jnp.float32),
            pltpu.VMEM((_NS, _BPC), jnp.float32),
            pltpu.VMEM((_NS, _BPC), jnp.int32),
            pltpu.VMEM_SHARED((_NS, _BPC), jnp.float32),
            pltpu.VMEM_SHARED((_NS, _BPC), jnp.int32),
            pltpu.SemaphoreType.DMA,
            pltpu.SemaphoreType.DMA,
        ],
    )(_sc_body)
    return kern(inp_t, rs)


def _tc_body(xt_ref, tab_ref, o_ref):
    x = xt_ref[...]
    rs = jnp.sum(tab_ref[...], axis=1)
    m = x != 0
    sel = jnp.sum(jnp.where(m, rs[:, None], jnp.float32(0.0)), axis=0)
    cnt = jnp.sum(m, axis=0).astype(jnp.float32)
    scal = sel / (jnp.maximum(cnt, 1.0) * jnp.float32(_DIM))
    o_ref[...] = jnp.broadcast_to(scal[:, None], (_BB, _DIM))


def _tc_part(xt, table):
    return pl.pallas_call(
        _tc_body,
        grid=(_TCB // _BB,),
        in_specs=[
            pl.BlockSpec((_VOCAB, _BB), lambda i: (0, _SCB // _BB + i)),
            pl.BlockSpec((_VOCAB, _DIM), lambda i: (0, 0)),
        ],
        out_specs=pl.BlockSpec((_BB, _DIM), lambda i: (i, 0)),
        out_shape=jax.ShapeDtypeStruct((_TCB, _DIM), jnp.float32),
    )(xt, table)


def kernel(input, table):
    xt = input.T
    rs = _row_sums(table)
    sc_out = _sc_main(xt, rs)
    tc_out = _tc_part(xt, table)
    return jnp.concatenate([sc_out, tc_out], axis=0)


# speedup vs baseline: 1.1463x; 1.0037x over previous
import functools

import jax
import jax.numpy as jnp
from jax import lax
from jax.experimental import pallas as pl
from jax.experimental.pallas import tpu as pltpu
from jax.experimental.pallas import tpu_sc as plsc

_BATCH = 4096
_VOCAB = 1000
_DIM = 128

_NC = 2
_NS = 16
_SCB = 1024
_TCB = _BATCH - _SCB
_BB = 512
_BPC = _SCB // _NC
_NCG = _BPC // 16
_OPT = _BPC // _NS
_TROWS = _VOCAB // 8
_NBUF = 2


def _row_sums_body(t_ref, o_ref):
    o_ref[...] = jnp.sum(t_ref[...], axis=1)


def _row_sums(table):
    return pl.pallas_call(
        _row_sums_body,
        out_shape=jax.ShapeDtypeStruct((_VOCAB,), jnp.float32),
    )(table)


def _sc_body(in_hbm, rs_hbm, out_hbm, rsbuf, inbufA, inbufB, acc_s, acc_c,
             outbuf, tmp_s, tmp_c, sh_s, sh_c, sem0, sem1):
    cid = lax.axis_index("c")
    sid = lax.axis_index("s")
    col0 = cid * _BPC
    ntr = jnp.where(sid < _TROWS - 7 * _NS, 8, 7)

    zf = jnp.zeros((16,), jnp.float32)
    zi = jnp.zeros((16,), jnp.int32)
    onef = jnp.ones((16,), jnp.float32)
    lane = lax.iota(jnp.int32, 16)

    pltpu.sync_copy(rs_hbm, rsbuf)

    def zero_body(i, carry):
        acc_s[pl.ds(i * 16, 16)] = zf
        acc_c[pl.ds(i * 16, 16)] = zi
        return carry

    lax.fori_loop(0, _NCG, zero_body, 0)

    inbufs = (inbufA, inbufB)
    sems = (sem0, sem1)

    def _fire(t, b):
        tr = sid + t * _NS
        pltpu.make_async_copy(
            in_hbm.at[pl.ds(tr * 8, 8), pl.ds(col0, _BPC)],
            inbufs[b],
            sems[b],
        ).start()

    def _drain(b):
        pltpu.make_async_copy(
            in_hbm.at[pl.ds(0, 8), pl.ds(col0, _BPC)],
            inbufs[b],
            sems[b],
        ).wait()

    def _compute(t, b):
        ib = inbufs[b]
        tr8 = (sid + t * _NS) * 8
        rsvs = [
            plsc.load_gather(rsbuf, [jnp.full((16,), tr8 + g, jnp.int32)])
            for g in range(8)
        ]

        def c_body(c):
            a_s = acc_s[pl.ds(c * 16, 16)]
            a_c = acc_c[pl.ds(c * 16, 16)]
            for g in range(8):
                x = ib[g, pl.ds(c * 16, 16)]
                a_c = a_c + x
                a_s = a_s + x.astype(jnp.float32) * rsvs[g]
            acc_s[pl.ds(c * 16, 16)] = a_s
            acc_c[pl.ds(c * 16, 16)] = a_c

        plsc.parallel_loop(0, _NCG, 1, unroll=2)(c_body)

    for b in range(_NBUF):
        _fire(b, b)

    def pair_body(p, carry):
        for b in range(_NBUF):
            t = p * _NBUF + b

            @pl.when(t < ntr)
            def _():
                _drain(b)
                _compute(t, b)

                @pl.when(t + _NBUF < ntr)
                def _():
                    _fire(t + _NBUF, b)

        return carry

    lax.fori_loop(0, 4, pair_body, 0)

    pltpu.sync_copy(acc_s, sh_s.at[sid])
    pltpu.sync_copy(acc_c, sh_c.at[sid])
    plsc.subcore_barrier()

    pltpu.sync_copy(sh_s, tmp_s)
    pltpu.sync_copy(sh_c, tmp_c)

    inv_d = jnp.float32(1.0 / _DIM)
    vecs = []
    for cg in range(_OPT // 16):
        tot_s = zf
        tot_c = zi
        for t in range(_NS):
            tot_s = tot_s + tmp_s[t, pl.ds(sid * _OPT + cg * 16, 16)]
            tot_c = tot_c + tmp_c[t, pl.ds(sid * _OPT + cg * 16, 16)]
        vecs.append(
            tot_s * inv_d
            / jnp.maximum(tot_c.astype(jnp.float32), onef))
    rows = [cg * 16 + lane for cg in range(_OPT // 16)]

    def d_body(d, carry):
        dsplat = jnp.full((16,), d, jnp.int32)
        for cg in range(_OPT // 16):
            plsc.store_scatter(outbuf, [rows[cg], dsplat], vecs[cg])
        return carry

    lax.fori_loop(0, _DIM, d_body, 0)

    base = col0 + sid * _OPT
    pltpu.sync_copy(outbuf, out_hbm.at[pl.ds(base, _OPT)])


def _sc_main(inp_t, rs):
    mesh = plsc.VectorSubcoreMesh(core_axis_name="c", subcore_axis_name="s")
    kern = functools.partial(
        pl.kernel,
        out_type=jax.ShapeDtypeStruct((_SCB, _DIM), jnp.float32),
        mesh=mesh,
        compiler_params=pltpu.CompilerParams(needs_layout_passes=False),
        scratch_types=[
            pltpu.VMEM((_VOCAB,), jnp.float32),
            pltpu.VMEM((8, _BPC), jnp.int32),
            pltpu.VMEM((8, _BPC), jnp.int32),
            pltpu.VMEM((_BPC,), jnp.float32),
            pltpu.VMEM((_BPC,), jnp.int32),
            pltpu.VMEM((_OPT, _DIM), jnp.float32),
            pltpu.VMEM((_NS, _BPC), jnp.float32),
            pltpu.VMEM((_NS, _BPC), jnp.int32),
            pltpu.VMEM_SHARED((_NS, _BPC), jnp.float32),
            pltpu.VMEM_SHARED((_NS, _BPC), jnp.int32),
            pltpu.SemaphoreType.DMA,
            pltpu.SemaphoreType.DMA,
        ],
    )(_sc_body)
    return kern(inp_t, rs)


def _tc_body(xt_ref, tab_ref, o_ref):
    x = xt_ref[...]
    rs = jnp.sum(tab_ref[...], axis=1)
    xf = x.astype(jnp.float32)
    sel = jnp.dot(rs, xf, preferred_element_type=jnp.float32)
    cnt = jnp.sum(xf, axis=0)
    scal = sel / (jnp.maximum(cnt, 1.0) * jnp.float32(_DIM))
    o_ref[...] = jnp.broadcast_to(scal[:, None], (_BB, _DIM))


def _tc_part(xt, table):
    return pl.pallas_call(
        _tc_body,
        grid=(_TCB // _BB,),
        in_specs=[
            pl.BlockSpec((_VOCAB, _BB), lambda i: (0, _SCB // _BB + i)),
            pl.BlockSpec((_VOCAB, _DIM), lambda i: (0, 0)),
        ],
        out_specs=pl.BlockSpec((_BB, _DIM), lambda i: (i, 0)),
        out_shape=jax.ShapeDtypeStruct((_TCB, _DIM), jnp.float32),
    )(xt, table)


def kernel(input, table):
    xt = input.T
    rs = _row_sums(table)
    sc_out = _sc_main(xt, rs)
    tc_out = _tc_part(xt, table)
    return jnp.concatenate([sc_out, tc_out], axis=0)
